# baseline (speedup 1.0000x reference)
import functools

import jax
import jax.numpy as jnp
from jax import lax
from jax.experimental import pallas as pl
from jax.experimental.pallas import tpu as pltpu
from jax.experimental.pallas import tpu_sc as plsc

_BATCH = 16384
_HIST = 200
_D = 16
_NW = 32
_SUB = 512
_NSTR = _SUB // 128
_SUBS_PER_UNIT = 8
_UNITS_PER_W = 25
_NITER = _UNITS_PER_W * _SUBS_PER_UNIT
_QTR = _BATCH // 4
_BHI = _BATCH // 8

_mesh = plsc.VectorSubcoreMesh(core_axis_name="c", subcore_axis_name="s")


@functools.partial(
    pl.kernel,
    mesh=_mesh,
    out_type=(
        jax.ShapeDtypeStruct((_HIST, _BHI, 128), jnp.float32),
        jax.ShapeDtypeStruct((_HIST, _BHI, 128), jnp.float32),
    ),
    scratch_types=[
        pltpu.VMEM((2, _SUB), jnp.int32),
        pltpu.VMEM((2, _SUB, _D), jnp.float32),
        pltpu.VMEM((2, _SUB, _D), jnp.float32),
        pltpu.VMEM((2, _SUB // 8, 128), jnp.float32),
        pltpu.VMEM((2, _SUB // 8, 128), jnp.float32),
        pltpu.SemaphoreType.DMA,
        pltpu.SemaphoreType.DMA,
        pltpu.SemaphoreType.DMA,
    ],
    compiler_params=pltpu.CompilerParams(use_tc_tiling_on_sc=False),
)
def _gather2(xt_hbm, mu_hbm, lv_hbm, out_mu, out_lv,
             idx_v, mu_rows, lv_rows, mu_pk, lv_pk, sem_idx, sem_g, sem_w):
    cid = lax.axis_index("c")
    sid = lax.axis_index("s")
    wid = sid * 2 + cid
    u0 = wid * _UNITS_PER_W

    def coords(k):
        u = u0 + k // _SUBS_PER_UNIT
        h = u // 4
        q = u % 4
        boff = q * _QTR + (k % _SUBS_PER_UNIT) * _SUB
        return h, boff

    def stage_idx(k, slot):
        h, boff = coords(k)
        return pltpu.async_copy(xt_hbm.at[h, pl.ds(boff, _SUB)],
                                idx_v.at[slot], sem_idx)

    def gather_descs(slot):
        descs = []
        for t in range(_NSTR):
            isl = pl.ds(t * 128, 128)
            descs.append((mu_hbm.at[idx_v.at[slot, isl]],
                          mu_rows.at[slot, isl]))
            descs.append((lv_hbm.at[idx_v.at[slot, isl]],
                          lv_rows.at[slot, isl]))
        return descs

    def fire_gathers(slot):
        for src, dst in gather_descs(slot):
            pltpu.async_copy(src, dst, sem_g)

    def drain_gathers(slot):
        for src, dst in gather_descs(slot):
            pltpu.make_async_copy(src, dst, sem_g).wait()

    def write_descs(k, slot):
        h, boff = coords(k)
        bh0 = boff // 8
        return [(mu_pk.at[slot], out_mu.at[h, pl.ds(bh0, _SUB // 8)]),
                (lv_pk.at[slot], out_lv.at[h, pl.ds(bh0, _SUB // 8)])]

    def repack(slot):
        def rbody(g, carry):
            for rsub in range(8):
                r = g * 8 + rsub
                mu_pk[slot, g, pl.ds(rsub * _D, _D)] = mu_rows[slot, r]
                lv_pk[slot, g, pl.ds(rsub * _D, _D)] = lv_rows[slot, r]
            return carry

        lax.fori_loop(0, _SUB // 8, rbody, 0)

    stage_idx(0, 0).wait()
    fire_gathers(0)
    stage_idx(1, 1)

    def body(k, carry):
        a = k % 2
        b = 1 - a
        has1 = k + 1 < _NITER
        has2 = k + 2 < _NITER

        drain_gathers(a)

        @pl.when(has1)
        def _():
            h1, boff1 = coords(k + 1)
            pltpu.make_async_copy(xt_hbm.at[h1, pl.ds(boff1, _SUB)],
                                  idx_v.at[b], sem_idx).wait()
            fire_gathers(b)

        @pl.when(has2)
        def _():
            stage_idx(k + 2, a)

        @pl.when(k >= 2)
        def _():
            for src, dst in write_descs(k - 2, a):
                pltpu.make_async_copy(src, dst, sem_w).wait()

        repack(a)
        for src, dst in write_descs(k, a):
            pltpu.async_copy(src, dst, sem_w)

        return carry

    lax.fori_loop(0, _NITER, body, 0)

    for kk in (_NITER - 2, _NITER - 1):
        for src, dst in write_descs(kk, kk % 2):
            pltpu.make_async_copy(src, dst, sem_w).wait()


def _tr_body(x_ref, o_ref):
    x3 = x_ref[0].reshape(_BHI, 8, _D)
    o_ref[0] = x3.transpose(2, 0, 1).reshape(_D, _BATCH)


_transpose_h = pl.pallas_call(
    _tr_body,
    grid=(_HIST,),
    in_specs=[pl.BlockSpec((1, _BHI, 128), lambda h: (h, 0, 0))],
    out_specs=pl.BlockSpec((1, _D, _BATCH), lambda h: (h, 0, 0)),
    out_shape=jax.ShapeDtypeStruct((_HIST, _D, _BATCH), jnp.float32),
)


def kernel(x, embed_mu, embed_logvar):
    xt = jnp.swapaxes(x.astype(jnp.int32), 0, 1)
    g_mu, g_lv = _gather2(xt, embed_mu, embed_logvar)
    mu = _transpose_h(g_mu).transpose(2, 0, 1)
    logvar = _transpose_h(g_lv).transpose(2, 0, 1)
    return (mu, mu, logvar)


# speedup vs baseline: 1.5848x; 1.5848x over previous
import functools

import jax
import jax.numpy as jnp
from jax import lax
from jax.experimental import pallas as pl
from jax.experimental.pallas import tpu as pltpu
from jax.experimental.pallas import tpu_sc as plsc

_BATCH = 16384
_HIST = 200
_D = 16
_NW = 32
_B_PER_W = _BATCH // _NW
_NB = 4
_NITER = _B_PER_W // _NB
_SPLITS = ((0, 128), (128, 72))

_mesh = plsc.VectorSubcoreMesh(core_axis_name="c", subcore_axis_name="s")


@functools.partial(
    pl.kernel,
    mesh=_mesh,
    out_type=jax.ShapeDtypeStruct((_BATCH, _HIST, _D), jnp.float32),
    scratch_types=[
        pltpu.VMEM((2, _NB, _HIST), jnp.int32),
        pltpu.VMEM((2, _NB, _HIST, _D), jnp.float32),
        pltpu.SemaphoreType.DMA,
        pltpu.SemaphoreType.DMA,
        pltpu.SemaphoreType.DMA,
    ],
    compiler_params=pltpu.CompilerParams(use_tc_tiling_on_sc=False),
)
def _gather1(x_hbm, mu_hbm, out_mu,
             idx_v, mu_rows, sem_idx, sem_g, sem_w):
    cid = lax.axis_index("c")
    sid = lax.axis_index("s")
    wid = sid * 2 + cid
    b0 = wid * _B_PER_W

    def fire_gathers(slot):
        for i in range(_NB):
            for off, ln in _SPLITS:
                pltpu.async_copy(mu_hbm.at[idx_v.at[slot, i, pl.ds(off, ln)]],
                                 mu_rows.at[slot, i, pl.ds(off, ln)], sem_g)

    def drain_gathers(slot):
        for i in range(_NB):
            for off, ln in _SPLITS:
                pltpu.make_async_copy(
                    mu_hbm.at[idx_v.at[slot, i, pl.ds(off, ln)]],
                    mu_rows.at[slot, i, pl.ds(off, ln)], sem_g).wait()

    pltpu.sync_copy(x_hbm.at[pl.ds(b0, _NB)], idx_v.at[0])
    fire_gathers(0)

    def body(j, carry):
        s = j % 2
        ns = 1 - s
        b = b0 + j * _NB
        has_next = j + 1 < _NITER

        @pl.when(has_next)
        def _():
            pltpu.async_copy(x_hbm.at[pl.ds(b + _NB, _NB)],
                             idx_v.at[ns], sem_idx)

        drain_gathers(s)
        pltpu.async_copy(mu_rows.at[s], out_mu.at[pl.ds(b, _NB)], sem_w)

        @pl.when(has_next & (j > 0))
        def _():
            pltpu.make_async_copy(mu_rows.at[ns],
                                  out_mu.at[pl.ds(b - _NB, _NB)],
                                  sem_w).wait()

        @pl.when(has_next)
        def _():
            pltpu.make_async_copy(x_hbm.at[pl.ds(b + _NB, _NB)],
                                  idx_v.at[ns], sem_idx).wait()
            fire_gathers(ns)

        return carry

    lax.fori_loop(0, _NITER, body, 0)

    for jj in (_NITER - 2, _NITER - 1):
        s = jj % 2
        b = b0 + jj * _NB
        pltpu.make_async_copy(mu_rows.at[s],
                              out_mu.at[pl.ds(b, _NB)], sem_w).wait()


def kernel(x, embed_mu, embed_logvar):
    x32 = x.astype(jnp.int32)
    mu = _gather1(x32, embed_mu)
    logvar = _gather1(x32, embed_logvar)
    return (mu, mu, logvar)
